# baseline (device time: 214410 ns/iter reference)
import jax
import jax.numpy as jnp
from jax import lax
from jax.experimental import pallas as pl
from jax.experimental.pallas import tpu as pltpu

N_DEV = 8
N_EXP = 32
E_LOC = 4
CAP = 409
T = 2048
D = 512
H = 1024

R_HOPS = 4
L_HOPS = 3


def _cumsum0(a, n_rows):
    sh = 1
    while sh < n_rows:
        z = jnp.zeros((sh, a.shape[1]), a.dtype)
        a = a + jnp.concatenate([z, a[:-sh, :]], axis=0)
        sh *= 2
    return a


def kernel(x, router_W, route_idx, expert_W):
    del router_W
    x_bf = x.astype(jnp.bfloat16)
    w_bf = expert_W.astype(jnp.bfloat16)

    def body(x_ref, ridx_ref, w_ref, out_ref,
             rbuf, lbuf, cnts,
             rssem, rrsem, lssem, lrsem,
             cssem, crsem):
        me = lax.axis_index("i")
        left = lax.rem(me - 1 + N_DEV, N_DEV)
        right = lax.rem(me + 1, N_DEV)

        def copy(src, dst, ssem, rsem, dev):
            return pltpu.make_async_remote_copy(
                src_ref=src, dst_ref=dst, send_sem=ssem, recv_sem=rsem,
                device_id=(dev,), device_id_type=pl.DeviceIdType.MESH,
            )

        ridx = ridx_ref[:, :]
        eids = lax.broadcasted_iota(jnp.int32, (T, N_EXP), 1)
        oh = (ridx == eids).astype(jnp.int32)
        cnts[pl.ds(me, 1), :] = jnp.sum(oh, axis=0, keepdims=True)

        bar = pltpu.get_barrier_semaphore()
        for j in range(1, N_DEV):
            pl.semaphore_signal(bar, inc=1,
                                device_id=(lax.rem(me + j, N_DEV),),
                                device_id_type=pl.DeviceIdType.MESH)
        pl.semaphore_wait(bar, N_DEV - 1)

        myrow = cnts.at[pl.ds(me, 1)]
        crdmas = [copy(myrow, myrow, cssem.at[j - 1], crsem.at[j - 1],
                       lax.rem(me + j, N_DEV))
                  for j in range(1, N_DEV)]
        for rd in crdmas:
            rd.start()
        for rd in crdmas:
            rd.wait()

        step0 = [copy(w_ref.at[pl.ds(0, 2)], rbuf.at[0],
                      rssem.at[0], rrsem.at[0], right),
                 copy(w_ref.at[pl.ds(2, 2)], lbuf.at[0],
                      lssem.at[0], lrsem.at[0], left)]
        for rd in step0:
            rd.start()

        srows = lax.broadcasted_iota(jnp.int32, (N_DEV, N_EXP), 0)
        prefix = jnp.sum(jnp.where(srows < me, cnts[:, :], 0),
                         axis=0, keepdims=True)
        excl = _cumsum0(oh, T) - oh
        local_rank = jnp.sum(oh * excl, axis=1, keepdims=True)
        tok_prefix = jnp.sum(oh * prefix, axis=1, keepdims=True)
        keep = (local_rank + tok_prefix) < CAP

        def accum_gran(is_first, e0, wgran):
            masks = [jnp.logical_and(ridx == e0 + k, keep)
                     .astype(jnp.bfloat16) for k in range(2)]
            w_cat = wgran.reshape(2 * D, H)
            q = T // 4
            for t0 in range(0, T, q):
                sl = pl.ds(t0, q)
                xm_cat = jnp.concatenate(
                    [x_ref[sl, :] * m[t0:t0 + q] for m in masks], axis=1)
                contrib = jnp.dot(xm_cat, w_cat,
                                  preferred_element_type=jnp.float32)
                if is_first:
                    out_ref[sl, :] = contrib
                else:
                    out_ref[sl, :] = out_ref[sl, :] + contrib

        def recv_gran(rs, right_chain):
            d = rs + 1 if rs < 4 else rs - 3
            koff = (0 if rs < 4 else 2) if right_chain else (2 if rs < 4 else 0)
            blk = lax.rem((me - d if right_chain else me + d) + 2 * N_DEV,
                          N_DEV)
            buf = rbuf if right_chain else lbuf
            accum_gran(False, blk * E_LOC + koff, buf[rs % 4])

        accum_gran(True, me * E_LOC, w_ref[pl.ds(0, 2)])
        accum_gran(False, me * E_LOC + 2, w_ref[pl.ds(2, 2)])
        for rd in step0:
            rd.wait()

        for s in range(1, 7):
            rsrc = w_ref.at[pl.ds(2, 2)] if s == 4 else rbuf.at[(s - 1) % 4]
            lsrc = w_ref.at[pl.ds(0, 2)] if s == 4 else lbuf.at[(s - 1) % 4]
            rdmas = [copy(rsrc, rbuf.at[s % 4],
                          rssem.at[s], rrsem.at[s], right),
                     copy(lsrc, lbuf.at[s % 4],
                          lssem.at[s], lrsem.at[s], left)]
            for rd in rdmas:
                rd.start()
            recv_gran(s - 1, True)
            recv_gran(s - 1, False)
            for rd in rdmas:
                rd.wait()
        recv_gran(6, True)
        recv_gran(6, False)

    return pl.pallas_call(
        body,
        out_shape=jax.ShapeDtypeStruct((T, H), jnp.float32),
        in_specs=[
            pl.BlockSpec(memory_space=pltpu.VMEM),
            pl.BlockSpec(memory_space=pltpu.VMEM),
            pl.BlockSpec(memory_space=pltpu.VMEM),
        ],
        out_specs=pl.BlockSpec(memory_space=pltpu.VMEM),
        scratch_shapes=[
            pltpu.VMEM((4, 2, D, H), jnp.bfloat16),
            pltpu.VMEM((4, 2, D, H), jnp.bfloat16),
            pltpu.VMEM((N_DEV, N_EXP), jnp.int32),
            pltpu.SemaphoreType.DMA((7,)),
            pltpu.SemaphoreType.DMA((7,)),
            pltpu.SemaphoreType.DMA((7,)),
            pltpu.SemaphoreType.DMA((7,)),
            pltpu.SemaphoreType.DMA((N_DEV - 1,)),
            pltpu.SemaphoreType.DMA((N_DEV - 1,)),
        ],
        compiler_params=pltpu.CompilerParams(
            collective_id=0,
            vmem_limit_bytes=63 * 1024 * 1024,
        ),
    )(x_bf, route_idx, w_bf)


# device time: 207764 ns/iter; 1.0320x vs baseline; 1.0320x over previous
import jax
import jax.numpy as jnp
from jax import lax
from jax.experimental import pallas as pl
from jax.experimental.pallas import tpu as pltpu

N_DEV = 8
N_EXP = 32
E_LOC = 4
CAP = 409
T = 2048
D = 512
H = 1024

R_HOPS = 4
L_HOPS = 3


def _cumsum0(a, n_rows):
    sh = 1
    while sh < n_rows:
        z = jnp.zeros((sh, a.shape[1]), a.dtype)
        a = a + jnp.concatenate([z, a[:-sh, :]], axis=0)
        sh *= 2
    return a


def kernel(x, router_W, route_idx, expert_W):
    del router_W
    x_bf = x.astype(jnp.bfloat16)
    w_bf = expert_W.astype(jnp.bfloat16)

    def body(x_ref, ridx_ref, w_ref, out_ref,
             rbuf, lbuf, cnts,
             rssem, rrsem, lssem, lrsem,
             cssem, crsem):
        me = lax.axis_index("i")
        left = lax.rem(me - 1 + N_DEV, N_DEV)
        right = lax.rem(me + 1, N_DEV)

        def copy(src, dst, ssem, rsem, dev):
            return pltpu.make_async_remote_copy(
                src_ref=src, dst_ref=dst, send_sem=ssem, recv_sem=rsem,
                device_id=(dev,), device_id_type=pl.DeviceIdType.MESH,
            )

        ridx = ridx_ref[:, :]
        eids = lax.broadcasted_iota(jnp.int32, (T, N_EXP), 1)
        oh = (ridx == eids).astype(jnp.int32)
        cnts[pl.ds(me, 1), :] = jnp.sum(oh, axis=0, keepdims=True)

        bar = pltpu.get_barrier_semaphore()
        for j in range(1, N_DEV):
            pl.semaphore_signal(bar, inc=1,
                                device_id=(lax.rem(me + j, N_DEV),),
                                device_id_type=pl.DeviceIdType.MESH)
        pl.semaphore_wait(bar, N_DEV - 1)

        myrow = cnts.at[pl.ds(me, 1)]
        crdmas = [copy(myrow, myrow, cssem.at[j - 1], crsem.at[j - 1],
                       lax.rem(me + j, N_DEV))
                  for j in range(1, N_DEV)]
        for rd in crdmas:
            rd.start()
        for rd in crdmas:
            rd.wait()

        r1 = [copy(w_ref, rbuf.at[0], rssem.at[0], rrsem.at[0], right),
              copy(w_ref, lbuf.at[0], lssem.at[0], lrsem.at[0], left)]
        for rd in r1:
            rd.start()

        srows = lax.broadcasted_iota(jnp.int32, (N_DEV, N_EXP), 0)
        prefix = jnp.sum(jnp.where(srows < me, cnts[:, :], 0),
                         axis=0, keepdims=True)
        excl = _cumsum0(oh, T) - oh
        local_rank = jnp.sum(oh * excl, axis=1, keepdims=True)
        tok_prefix = jnp.sum(oh * prefix, axis=1, keepdims=True)
        keep = (local_rank + tok_prefix) < CAP

        def accum(is_first, e, w):
            m = jnp.logical_and(ridx == e, keep).astype(jnp.bfloat16)
            half = T // 2
            for t0 in (0, half):
                sl = pl.ds(t0, half)
                xm = x_ref[sl, :] * m[t0:t0 + half]
                contrib = jnp.dot(xm, w, preferred_element_type=jnp.float32)
                if is_first:
                    out_ref[sl, :] = contrib
                else:
                    out_ref[sl, :] = out_ref[sl, :] + contrib

        def accum_block(is_first, origin, wblock):
            o = lax.rem(origin + 2 * N_DEV, N_DEV)
            for k in range(E_LOC):
                accum(is_first and k == 0, o * E_LOC + k, wblock[k])

        accum_block(True, me, w_ref)
        for rd in r1:
            rd.wait()
        for r in (2, 3):
            rdmas = [copy(rbuf.at[r - 2], rbuf.at[r - 1],
                          rssem.at[r - 1], rrsem.at[r - 1], right),
                     copy(lbuf.at[r - 2], lbuf.at[r - 1],
                          lssem.at[r - 1], lrsem.at[r - 1], left)]
            for rd in rdmas:
                rd.start()
            accum_block(False, me - (r - 1), rbuf[r - 2])
            accum_block(False, me + (r - 1), lbuf[r - 2])
            for rd in rdmas:
                rd.wait()
        rdmas = [copy(rbuf.at[2, pl.ds(0, 2)], lbuf.at[0, pl.ds(0, 2)],
                      rssem.at[3], rrsem.at[3], right),
                 copy(lbuf.at[2, pl.ds(2, 2)], lbuf.at[0, pl.ds(2, 2)],
                      lssem.at[3], lrsem.at[3], left)]
        for rd in rdmas:
            rd.start()
        accum_block(False, me - 3, rbuf[2])
        accum_block(False, me + 3, lbuf[2])
        for rd in rdmas:
            rd.wait()
        accum_block(False, me - 4, lbuf[0])

    return pl.pallas_call(
        body,
        out_shape=jax.ShapeDtypeStruct((T, H), jnp.float32),
        in_specs=[
            pl.BlockSpec(memory_space=pltpu.VMEM),
            pl.BlockSpec(memory_space=pltpu.VMEM),
            pl.BlockSpec(memory_space=pltpu.VMEM),
        ],
        out_specs=pl.BlockSpec(memory_space=pltpu.VMEM),
        scratch_shapes=[
            pltpu.VMEM((R_HOPS - 1, E_LOC, D, H), jnp.bfloat16),
            pltpu.VMEM((L_HOPS, E_LOC, D, H), jnp.bfloat16),
            pltpu.VMEM((N_DEV, N_EXP), jnp.int32),
            pltpu.SemaphoreType.DMA((R_HOPS,)),
            pltpu.SemaphoreType.DMA((R_HOPS,)),
            pltpu.SemaphoreType.DMA((R_HOPS,)),
            pltpu.SemaphoreType.DMA((R_HOPS,)),
            pltpu.SemaphoreType.DMA((N_DEV - 1,)),
            pltpu.SemaphoreType.DMA((N_DEV - 1,)),
        ],
        compiler_params=pltpu.CompilerParams(
            collective_id=0,
            vmem_limit_bytes=63 * 1024 * 1024,
        ),
    )(x_bf, route_idx, w_bf)
